# Initial kernel scaffold; baseline (speedup 1.0000x reference)
import jax
import jax.numpy as jnp
from jax.experimental import pallas as pl


def kernel(x, edge_index, batch, params):
    raise NotImplementedError("write your pallas kernel here")


# speedup vs baseline: 1.8756x; 1.8756x over previous
import functools

import jax
import jax.numpy as jnp
from jax import lax
from jax.experimental import pallas as pl
from jax.experimental.pallas import tpu as pltpu
from jax.experimental.pallas import tpu_sc as plsc

N_NODES = 10000
HID = 300
SLAB = 128
N_LAYERS = 5

CHUNK = 128
N_SUBCORES = 16
N_CORES = 2
ROWS_PER_TILE = 632
ROWS_LAST = N_NODES - ROWS_PER_TILE * (N_SUBCORES - 1)
AGG_ROWS = N_NODES + 8


def _make_agg_kernel(n_slabs, chunks_per_worker):
    mesh = plsc.VectorSubcoreMesh(core_axis_name="c", subcore_axis_name="s")

    @functools.partial(
        pl.kernel,
        mesh=mesh,
        out_type=jax.ShapeDtypeStruct((2 * n_slabs * N_NODES, SLAB), jnp.float32),
        scratch_types=[
            pltpu.VMEM((CHUNK,), jnp.int32),
            pltpu.VMEM((CHUNK,), jnp.int32),
            pltpu.VMEM((CHUNK,), jnp.int32),
            pltpu.VMEM((CHUNK, SLAB), jnp.float32),
            pltpu.VMEM_SHARED((AGG_ROWS, SLAB), jnp.float32),
            pltpu.SemaphoreType.DMA,
        ],
    )
    def agg_kernel(h_hbm, src_hbm, dst_hbm, zeros_hbm, out_hbm,
                   src_v, dst_v, gidx_v, gbuf, acc, sem):
        c = lax.axis_index("c")
        s = lax.axis_index("s")
        w = c * N_SUBCORES + s

        for slab in range(n_slabs):
            @pl.when(s < N_SUBCORES - 1)
            def _zero_main():
                pltpu.sync_copy(
                    zeros_hbm, acc.at[pl.ds(s * ROWS_PER_TILE, ROWS_PER_TILE)])

            @pl.when(s == N_SUBCORES - 1)
            def _zero_tail():
                pltpu.sync_copy(
                    zeros_hbm.at[pl.ds(0, ROWS_LAST + 8)],
                    acc.at[pl.ds((N_SUBCORES - 1) * ROWS_PER_TILE,
                                 ROWS_LAST + 8)])

            plsc.subcore_barrier()

            def body(k, carry):
                off = (w * chunks_per_worker + k) * CHUNK
                pltpu.sync_copy(src_hbm.at[pl.ds(off, CHUNK)], src_v)
                pltpu.sync_copy(dst_hbm.at[pl.ds(off, CHUNK)], dst_v)
                row_off = slab * N_NODES
                for j in range(CHUNK // 16):
                    sl = pl.ds(j * 16, 16)
                    gidx_v[sl] = src_v[sl] + row_off
                pltpu.async_copy(h_hbm.at[gidx_v], gbuf, sem).wait()
                pltpu.sync_copy(gbuf, acc.at[dst_v], add=True)
                return carry

            lax.fori_loop(0, chunks_per_worker, body, 0)
            plsc.subcore_barrier()

            out_base = (c * n_slabs + slab) * N_NODES + s * ROWS_PER_TILE

            @pl.when(s < N_SUBCORES - 1)
            def _copy_main():
                pltpu.sync_copy(
                    acc.at[pl.ds(s * ROWS_PER_TILE, ROWS_PER_TILE)],
                    out_hbm.at[pl.ds(out_base, ROWS_PER_TILE)])

            @pl.when(s == N_SUBCORES - 1)
            def _copy_tail():
                pltpu.sync_copy(
                    acc.at[pl.ds((N_SUBCORES - 1) * ROWS_PER_TILE, ROWS_LAST)],
                    out_hbm.at[pl.ds(out_base, ROWS_LAST)])

            plsc.subcore_barrier()

    return agg_kernel


def _make_mlp_kernel(n_slabs_in, n_slabs_out, last, bn):
    g = N_NODES // bn
    if last:
        out_shape = jax.ShapeDtypeStruct((N_NODES, HID), jnp.float32)
        out_spec = pl.BlockSpec((bn, HID), lambda i: (i, 0))
        out_cols = HID
    else:
        out_shape = jax.ShapeDtypeStruct((n_slabs_out, N_NODES, SLAB), jnp.float32)
        out_spec = pl.BlockSpec((n_slabs_out, bn, SLAB), lambda i: (0, i, 0))
        out_cols = n_slabs_out * SLAB

    def mlp_kernel(h_ref, a_ref, w1_ref, b1_ref, w2_ref, b2_ref, o_ref):
        t = b1_ref[...]
        for k in range(n_slabs_in):
            m = h_ref[k] + a_ref[0, k] + a_ref[1, k]
            t = t + jnp.dot(m, w1_ref[k], preferred_element_type=jnp.float32)
        t = jnp.maximum(t, 0.0)
        o = jnp.dot(t, w2_ref[...], preferred_element_type=jnp.float32) + b2_ref[...]
        if last:
            o_ref[...] = o
        else:
            o = jnp.maximum(o, 0.0)
            for k in range(n_slabs_out):
                o_ref[k] = o[:, k * SLAB:(k + 1) * SLAB]

    return pl.pallas_call(
        mlp_kernel,
        grid=(g,),
        in_specs=[
            pl.BlockSpec((n_slabs_in, bn, SLAB), lambda i: (0, i, 0)),
            pl.BlockSpec((2, n_slabs_in, bn, SLAB), lambda i: (0, 0, i, 0)),
            pl.BlockSpec((n_slabs_in, SLAB, 2 * HID), lambda i: (0, 0, 0)),
            pl.BlockSpec((1, 2 * HID), lambda i: (0, 0)),
            pl.BlockSpec((2 * HID, out_cols), lambda i: (0, 0)),
            pl.BlockSpec((1, out_cols), lambda i: (0, 0)),
        ],
        out_specs=out_spec,
        out_shape=out_shape,
    )


def kernel(x, edge_index, batch, params):
    n, f_in = x.shape
    e = edge_index.shape[1]
    assert f_in == SLAB and n == N_NODES
    n_slabs_h = -(-HID // SLAB)

    unit = CHUNK * N_SUBCORES * N_CORES
    ep = -(-e // unit) * unit
    pad = ep - e
    src = jnp.concatenate([edge_index[0], jnp.zeros((pad,), jnp.int32)])
    dst = jnp.concatenate([edge_index[1], jnp.full((pad,), n, jnp.int32)])
    chunks_per_worker = ep // (CHUNK * N_SUBCORES * N_CORES)

    zeros_hbm = jnp.zeros((ROWS_PER_TILE, SLAB), jnp.float32)

    agg1 = _make_agg_kernel(1, chunks_per_worker)
    agg3 = _make_agg_kernel(n_slabs_h, chunks_per_worker)

    h = x.reshape(1, n, SLAB)
    for l in range(N_LAYERS):
        n_slabs_in = 1 if l == 0 else n_slabs_h
        last = l == N_LAYERS - 1

        scale = params['bn_g_%d' % l] / jnp.sqrt(1.0 + 1e-05)
        w2 = params['W2_%d' % l] * scale[None, :]
        b2 = params['b2_%d' % l] * scale + params['bn_b_%d' % l]

        w1 = params['W1_%d' % l]
        din = w1.shape[0]
        if n_slabs_in * SLAB > din:
            w1 = jnp.concatenate(
                [w1, jnp.zeros((n_slabs_in * SLAB - din, 2 * HID), jnp.float32)])
        w1 = w1.reshape(n_slabs_in, SLAB, 2 * HID)
        if not last:
            out_cols = n_slabs_h * SLAB
            w2 = jnp.concatenate(
                [w2, jnp.zeros((2 * HID, out_cols - HID), jnp.float32)], axis=1)
            b2 = jnp.concatenate([b2, jnp.zeros((out_cols - HID,), jnp.float32)])
        b1 = params['b1_%d' % l].reshape(1, 2 * HID)
        b2 = b2.reshape(1, -1)

        agg_fn = agg1 if l == 0 else agg3
        h_flat = h.reshape(n_slabs_in * n, SLAB)
        agg_flat = agg_fn(h_flat, src, dst, zeros_hbm)
        agg = agg_flat.reshape(2, n_slabs_in, n, SLAB)

        mlp = _make_mlp_kernel(n_slabs_in, n_slabs_h, last, bn=2000)
        h = mlp(h, agg, w1, b1, w2, b2)

    return h
